# baseline (speedup 1.0000x reference)
import functools

import jax
import jax.numpy as jnp
from jax import lax
from jax.experimental import pallas as pl
from jax.experimental.pallas import tpu as pltpu
from jax.experimental.pallas import tpu_sc as plsc

N = 10000
D = 128
NC = 2
NS = 16
NW = NC * NS
CHUNK = 64
NPAD = 10240
RPT = NPAD // NS
RB = 1000
NB = N // RB

def _mesh():
    return plsc.VectorSubcoreMesh(
        core_axis_name="c", subcore_axis_name="s",
        num_cores=NC, num_subcores=NS,
    )


def _zero_rows(buf, nrows, width):
    zeros = jnp.zeros((16,), jnp.float32)

    def body(j, _):
        for s in range(width // 16):
            buf[j, pl.ds(s * 16, 16)] = zeros
        return 0

    lax.fori_loop(0, nrows, body, 0)


def _zero_acc_slice(zbuf, acc_s, base, nrows):
    full, rem = nrows // CHUNK, nrows % CHUNK
    for k in range(full):
        pltpu.sync_copy(zbuf, acc_s.at[pl.ds(base + k * CHUNK, CHUNK)])
    if rem:
        pltpu.sync_copy(
            zbuf.at[pl.ds(0, rem)], acc_s.at[pl.ds(base + full * CHUNK, rem)]
        )


NBUF = 4
NSRC = 8
GRP = 8


def _make_agg_kernel(nchunk):
    assert nchunk % GRP == 0
    ng = nchunk // GRP
    scratch = [pltpu.VMEM((nchunk // 2, 2 * CHUNK), jnp.float32)]
    scratch += [pltpu.VMEM((CHUNK, D), jnp.float32) for _ in range(NBUF)]
    scratch += [pltpu.VMEM((NSRC, CHUNK), jnp.int32),
                pltpu.VMEM((NBUF, CHUNK), jnp.int32),
                pltpu.VMEM((8, 128), jnp.float32)]
    scratch += [pltpu.VMEM_SHARED((NPAD, D), jnp.float32)]
    scratch += [pltpu.SemaphoreType.DMA
                for _ in range(NBUF + NBUF + NSRC + NBUF)]

    @functools.partial(
        pl.kernel,
        mesh=_mesh(),
        out_type=jax.ShapeDtypeStruct((NC, NPAD, D), jnp.float32),
        scratch_types=scratch,
    )
    def agg_kernel(hs_hbm, src_hbm, dst_hbm, w_hbm, flag_hbm, out_hbm,
                   w_v, *rest):
        rows = rest[:NBUF]
        srcb_a = rest[NBUF]
        dstb_a = rest[NBUF + 1]
        flag_v = rest[NBUF + 2]
        acc_s = rest[NBUF + 3]
        sems = rest[NBUF + 4:]
        gsem = sems[:NBUF]
        ssem = sems[NBUF:2 * NBUF]
        isems = sems[2 * NBUF:2 * NBUF + NSRC]
        isemd = sems[2 * NBUF + NSRC:]
        c = lax.axis_index("c")
        s = lax.axis_index("s")
        wid = s * NC + c

        def src_dma(j, si):
            pltpu.async_copy(src_hbm.at[wid, j], srcb_a.at[si], isems[si])

        def src_wait(j, si):
            pltpu.make_async_copy(
                src_hbm.at[wid, j], srcb_a.at[si], isems[si]).wait()

        def dst_dma(j, b):
            pltpu.async_copy(dst_hbm.at[wid, j], dstb_a.at[b], isemd[b])

        def dst_wait(j, b):
            pltpu.make_async_copy(
                dst_hbm.at[wid, j], dstb_a.at[b], isemd[b]).wait()

        def gather(b, si):
            pltpu.async_copy(hs_hbm.at[srcb_a.at[si]], rows[b], gsem[b])

        def gwait(b, si):
            pltpu.make_async_copy(
                hs_hbm.at[srcb_a.at[si]], rows[b], gsem[b]).wait()

        def scatter(b):
            pltpu.async_copy(rows[b], acc_s.at[dstb_a.at[b]], ssem[b],
                             add=True)

        def swait(b):
            pltpu.make_async_copy(
                rows[b], acc_s.at[dstb_a.at[b]], ssem[b]).wait()

        pltpu.sync_copy(w_hbm.at[wid], w_v)
        pltpu.sync_copy(flag_hbm, flag_v)
        gmode = flag_v[c * 0, pl.ds(0, 16)][0] > 0.5
        _zero_rows(rows[0], CHUNK, D)
        base = s * RPT
        _zero_acc_slice(rows[0], acc_s, base, RPT)

        def prime():
            for j in range(NBUF):
                src_dma(j, j)
            for j in range(2):
                src_wait(j, j)
                gather(j, j)

        pl.when(gmode)(prime)
        for j in range(2):
            dst_dma(j, j)
        plsc.subcore_barrier()

        def group(g, _):
            for p in range(GRP):
                j = g * GRP + p
                b = p % NBUF
                b2 = (p + 2) % NBUF
                s2 = (p + 2) % NSRC
                s4 = (p + 4) % NSRC

                def drain():
                    swait(b2)

                def prefetch_src():
                    src_dma(j + 4, s4)

                def prefetch_dst():
                    dst_dma(j + 2, b2)

                def prefetch_rows():
                    src_wait(j + 2, s2)
                    gather(b2, s2)

                if p < 2:
                    pl.when(g >= 1)(drain)
                else:
                    drain()
                if p < NBUF:
                    pl.when(gmode)(prefetch_src)
                else:
                    pl.when(jnp.logical_and(gmode, g <= ng - 2))(prefetch_src)
                if p < GRP - 2:
                    prefetch_dst()
                    pl.when(gmode)(prefetch_rows)
                else:
                    pl.when(g <= ng - 2)(prefetch_dst)
                    pl.when(jnp.logical_and(gmode, g <= ng - 2))(prefetch_rows)

                pl.when(gmode)(lambda: gwait(b, p % NSRC))

                jh = g * (GRP // 2) + p // 2
                woff = (p % 2) * CHUNK

                def scale(gg, _):
                    wv16 = w_v[jh, pl.ds(woff + gg * 16, 16)]
                    for k in range(16):
                        wv = wv16[k]
                        jj = gg * 16 + k
                        for s8 in range(D // 16):
                            sl = pl.ds(s8 * 16, 16)
                            rows[b][jj, sl] = rows[b][jj, sl] * wv
                    return 0

                def fill(gg, _):
                    wv16 = w_v[jh, pl.ds(woff + gg * 16, 16)]
                    for k in range(16):
                        wrow = jnp.full((16,), wv16[k], jnp.float32)
                        jj = gg * 16 + k
                        for s8 in range(D // 16):
                            rows[b][jj, pl.ds(s8 * 16, 16)] = wrow
                    return 0

                def do_scale():
                    lax.fori_loop(0, CHUNK // 16, scale, 0, unroll=2)

                def do_fill():
                    lax.fori_loop(0, CHUNK // 16, fill, 0, unroll=2)

                pl.when(gmode)(do_scale)
                pl.when(jnp.logical_not(gmode))(do_fill)
                dst_wait(j, b)
                scatter(b)
            return 0

        lax.fori_loop(0, ng, group, 0)
        swait((nchunk - 2) % NBUF)
        swait((nchunk - 1) % NBUF)
        plsc.subcore_barrier()
        pltpu.sync_copy(
            acc_s.at[pl.ds(base, RPT)], out_hbm.at[c, pl.ds(base, RPT)]
        )

    return agg_kernel


def _dis_block(deg_ref):
    deg = deg_ref[0, :, :1] + deg_ref[1, :, :1] + 1.0
    return jnp.where(deg > 0, 1.0 / jnp.sqrt(deg), 0.0)


def _matmul_body(x_ref, w_ref, o_ref):
    o_ref[...] = jnp.dot(
        x_ref[...], w_ref[...], preferred_element_type=jnp.float32
    )


def _scale_body(deg_ref, h_ref, o_ref):
    o_ref[...] = h_ref[...] * _dis_block(deg_ref)


def _mid_body(deg_ref, acc_ref, hs_ref, b_ref, w_ref, o_ref):
    dis = _dis_block(deg_ref)
    t = (acc_ref[0] + acc_ref[1] + hs_ref[...]) * dis + b_ref[...]
    z = jnp.maximum(t, 0.0)
    o_ref[...] = (
        jnp.dot(z, w_ref[...], preferred_element_type=jnp.float32) * dis
    )


def _last_body(deg_ref, acc_ref, hs_ref, b_ref, w_ref, bl_ref, o_ref):
    dis = _dis_block(deg_ref)
    t = (acc_ref[0] + acc_ref[1] + hs_ref[...]) * dis + b_ref[...]
    z = jnp.maximum(t, 0.0)
    o_ref[...] = (
        jnp.dot(z, w_ref[...], preferred_element_type=jnp.float32)
        + bl_ref[...]
    )


_deg_spec = pl.BlockSpec((2, RB, D), lambda i: (0, i, 0))
_row_spec = pl.BlockSpec((RB, D), lambda i: (i, 0))
_acc_spec = pl.BlockSpec((2, RB, D), lambda i: (0, i, 0))
_mat_spec = pl.BlockSpec((D, D), lambda i: (0, 0))
_vec_spec = pl.BlockSpec((1, D), lambda i: (0, 0))


def _tc_matmul(x, W1):
    return pl.pallas_call(
        _matmul_body,
        grid=(NB,),
        in_specs=[_row_spec, _mat_spec],
        out_specs=_row_spec,
        out_shape=jax.ShapeDtypeStruct((N, D), jnp.float32),
    )(x, W1)


def _tc_scale(degacc, h):
    return pl.pallas_call(
        _scale_body,
        grid=(NB,),
        in_specs=[_deg_spec, _row_spec],
        out_specs=_row_spec,
        out_shape=jax.ShapeDtypeStruct((N, D), jnp.float32),
    )(degacc, h)


def _tc_mid(degacc, acc, hs, b, Wn):
    return pl.pallas_call(
        _mid_body,
        grid=(NB,),
        in_specs=[_deg_spec, _acc_spec, _row_spec, _vec_spec, _mat_spec],
        out_specs=_row_spec,
        out_shape=jax.ShapeDtypeStruct((N, D), jnp.float32),
    )(degacc, acc, hs, b.reshape(1, D), Wn)


def _tc_last(degacc, acc, hs, b, Wlp, blp):
    return pl.pallas_call(
        _last_body,
        grid=(NB,),
        in_specs=[_deg_spec, _acc_spec, _row_spec, _vec_spec, _mat_spec,
                  _vec_spec],
        out_specs=_row_spec,
        out_shape=jax.ShapeDtypeStruct((N, D), jnp.float32),
    )(degacc, acc, hs, b.reshape(1, D), Wlp, blp)


@jax.jit
def kernel(x, edge_index, edge_attr, W1, b1, W2, b2, W3, b3, Wl, bl):
    e = edge_attr.shape[0]
    src = edge_index[0].astype(jnp.int32)
    dst = edge_index[1].astype(jnp.int32)
    w = edge_attr.reshape(-1).astype(jnp.float32)

    nchunk = -(-e // (NW * CHUNK))
    nchunk = -(-nchunk // GRP) * GRP
    epad = NW * CHUNK * nchunk
    pad = epad - e
    pidx = jnp.arange(pad, dtype=jnp.int32) % N
    src3 = jnp.concatenate([src, pidx]).reshape(NW, nchunk, CHUNK)
    dst3 = jnp.concatenate([dst, pidx]).reshape(NW, nchunk, CHUNK)
    w3 = jnp.concatenate([w, jnp.zeros((pad,), jnp.float32)]).reshape(
        NW, nchunk // 2, 2 * CHUNK
    )

    agg_call = _make_agg_kernel(nchunk)
    fyes = jnp.ones((8, 128), jnp.float32)
    fno = jnp.zeros((8, 128), jnp.float32)

    h1 = _tc_matmul(x, W1)
    degacc = agg_call(x, src3, dst3, w3, fno)
    hs1 = _tc_scale(degacc, h1)
    acc1 = agg_call(hs1, src3, dst3, w3, fyes)
    hs2 = _tc_mid(degacc, acc1, hs1, b1, W2)
    acc2 = agg_call(hs2, src3, dst3, w3, fyes)
    hs3 = _tc_mid(degacc, acc2, hs2, b2, W3)
    acc3 = agg_call(hs3, src3, dst3, w3, fyes)

    Wlp = jnp.pad(Wl, ((0, 0), (0, D - Wl.shape[1])))
    blp = jnp.pad(bl, (0, D - bl.shape[0])).reshape(1, D)
    out = _tc_last(degacc, acc3, hs3, b3, Wlp, blp)
    return out[:, : Wl.shape[1]]


# speedup vs baseline: 1.1226x; 1.1226x over previous
import functools

import jax
import jax.numpy as jnp
from jax import lax
from jax.experimental import pallas as pl
from jax.experimental.pallas import tpu as pltpu
from jax.experimental.pallas import tpu_sc as plsc

N = 10000
D = 128
NC = 2
NS = 16
NW = NC * NS
CHUNK = 64
NPAD = 10240
RPT = NPAD // NS
RB = 1000
NB = N // RB

def _mesh():
    return plsc.VectorSubcoreMesh(
        core_axis_name="c", subcore_axis_name="s",
        num_cores=NC, num_subcores=NS,
    )


def _zero_rows(buf, nrows, width):
    zeros = jnp.zeros((16,), jnp.float32)

    def body(j, _):
        for s in range(width // 16):
            buf[j, pl.ds(s * 16, 16)] = zeros
        return 0

    lax.fori_loop(0, nrows, body, 0)


def _zero_acc_slice(zbuf, acc_s, base, nrows):
    full, rem = nrows // CHUNK, nrows % CHUNK
    for k in range(full):
        pltpu.sync_copy(zbuf, acc_s.at[pl.ds(base + k * CHUNK, CHUNK)])
    if rem:
        pltpu.sync_copy(
            zbuf.at[pl.ds(0, rem)], acc_s.at[pl.ds(base + full * CHUNK, rem)]
        )


NBUF = 4
NSRC = 8
GRP = 8


def _make_agg_kernel(nchunk):
    assert nchunk % GRP == 0
    ng = nchunk // GRP
    scratch = [pltpu.VMEM((nchunk // 2, 2 * CHUNK), jnp.float32)]
    scratch += [pltpu.VMEM((CHUNK, D), jnp.float32) for _ in range(NBUF)]
    scratch += [pltpu.VMEM((NSRC, CHUNK), jnp.int32),
                pltpu.VMEM((NBUF, CHUNK), jnp.int32)]
    scratch += [pltpu.VMEM_SHARED((NPAD, D), jnp.float32)]
    scratch += [pltpu.SemaphoreType.DMA
                for _ in range(NBUF + NBUF + NSRC + NBUF)]

    @functools.partial(
        pl.kernel,
        mesh=_mesh(),
        out_type=jax.ShapeDtypeStruct((NC, NPAD, D), jnp.float32),
        scratch_types=scratch,
    )
    def agg_kernel(hs_hbm, src_hbm, dst_hbm, w_hbm, out_hbm, w_v, *rest):
        rows = rest[:NBUF]
        srcb_a = rest[NBUF]
        dstb_a = rest[NBUF + 1]
        acc_s = rest[NBUF + 2]
        sems = rest[NBUF + 3:]
        gsem = sems[:NBUF]
        ssem = sems[NBUF:2 * NBUF]
        isems = sems[2 * NBUF:2 * NBUF + NSRC]
        isemd = sems[2 * NBUF + NSRC:]
        c = lax.axis_index("c")
        s = lax.axis_index("s")
        wid = s * NC + c

        def src_dma(j, si):
            pltpu.async_copy(src_hbm.at[wid, j], srcb_a.at[si], isems[si])

        def src_wait(j, si):
            pltpu.make_async_copy(
                src_hbm.at[wid, j], srcb_a.at[si], isems[si]).wait()

        def dst_dma(j, b):
            pltpu.async_copy(dst_hbm.at[wid, j], dstb_a.at[b], isemd[b])

        def dst_wait(j, b):
            pltpu.make_async_copy(
                dst_hbm.at[wid, j], dstb_a.at[b], isemd[b]).wait()

        def gather(b, si):
            pltpu.async_copy(hs_hbm.at[srcb_a.at[si]], rows[b], gsem[b])

        def gwait(b, si):
            pltpu.make_async_copy(
                hs_hbm.at[srcb_a.at[si]], rows[b], gsem[b]).wait()

        def scatter(b):
            pltpu.async_copy(rows[b], acc_s.at[dstb_a.at[b]], ssem[b],
                             add=True)

        def swait(b):
            pltpu.make_async_copy(
                rows[b], acc_s.at[dstb_a.at[b]], ssem[b]).wait()

        pltpu.sync_copy(w_hbm.at[wid], w_v)
        _zero_rows(rows[0], CHUNK, D)
        base = s * RPT
        _zero_acc_slice(rows[0], acc_s, base, RPT)
        for j in range(NBUF):
            src_dma(j, j)
        for j in range(2):
            dst_dma(j, j)
            src_wait(j, j)
            gather(j, j)
        plsc.subcore_barrier()

        def group(g, _):
            for p in range(GRP):
                j = g * GRP + p
                b = p % NBUF
                b2 = (p + 2) % NBUF
                s2 = (p + 2) % NSRC
                s4 = (p + 4) % NSRC

                def drain():
                    swait(b2)

                def prefetch_src():
                    src_dma(j + 4, s4)

                def prefetch_rows():
                    dst_dma(j + 2, b2)
                    src_wait(j + 2, s2)
                    gather(b2, s2)

                if p < 2:
                    pl.when(g >= 1)(drain)
                else:
                    drain()
                if p < NBUF:
                    prefetch_src()
                else:
                    pl.when(g <= ng - 2)(prefetch_src)
                if p < GRP - 2:
                    prefetch_rows()
                else:
                    pl.when(g <= ng - 2)(prefetch_rows)

                gwait(b, p % NSRC)

                jh = g * (GRP // 2) + p // 2
                woff = (p % 2) * CHUNK

                _dnums = lax.GatherDimensionNumbers(
                    offset_dims=(), collapsed_slice_dims=(0,),
                    start_index_map=(0,))

                def scale(gg, _):
                    wv16 = w_v[jh, pl.ds(woff + gg * 16, 16)]
                    for k in range(16):
                        kvec = jnp.full((16, 1), k, jnp.int32)
                        wrow = lax.gather(
                            wv16, kvec, _dnums, (1,),
                            mode=lax.GatherScatterMode.PROMISE_IN_BOUNDS)
                        jj = gg * 16 + k
                        for s8 in range(D // 16):
                            sl = pl.ds(s8 * 16, 16)
                            rows[b][jj, sl] = rows[b][jj, sl] * wrow
                    return 0

                lax.fori_loop(0, CHUNK // 16, scale, 0, unroll=2)
                dst_wait(j, b)
                scatter(b)
            return 0

        lax.fori_loop(0, ng, group, 0)
        swait((nchunk - 2) % NBUF)
        swait((nchunk - 1) % NBUF)
        plsc.subcore_barrier()
        pltpu.sync_copy(
            acc_s.at[pl.ds(base, RPT)], out_hbm.at[c, pl.ds(base, RPT)]
        )

    return agg_kernel


def _dis_block(deg_ref):
    deg = deg_ref[0, :, :1] + deg_ref[1, :, :1] + 1.0
    return jnp.where(deg > 0, 1.0 / jnp.sqrt(deg), 0.0)


def _matmul_body(x_ref, w_ref, o_ref):
    o_ref[...] = jnp.dot(
        x_ref[...], w_ref[...], preferred_element_type=jnp.float32
    )


def _scale_body(deg_ref, h_ref, o_ref):
    o_ref[...] = h_ref[...] * _dis_block(deg_ref)


def _mid_body(deg_ref, acc_ref, hs_ref, b_ref, w_ref, o_ref):
    dis = _dis_block(deg_ref)
    t = (acc_ref[0] + acc_ref[1] + hs_ref[...]) * dis + b_ref[...]
    z = jnp.maximum(t, 0.0)
    o_ref[...] = (
        jnp.dot(z, w_ref[...], preferred_element_type=jnp.float32) * dis
    )


def _last_body(deg_ref, acc_ref, hs_ref, b_ref, w_ref, bl_ref, o_ref):
    dis = _dis_block(deg_ref)
    t = (acc_ref[0] + acc_ref[1] + hs_ref[...]) * dis + b_ref[...]
    z = jnp.maximum(t, 0.0)
    o_ref[...] = (
        jnp.dot(z, w_ref[...], preferred_element_type=jnp.float32)
        + bl_ref[...]
    )


_deg_spec = pl.BlockSpec((2, RB, D), lambda i: (0, i, 0))
_row_spec = pl.BlockSpec((RB, D), lambda i: (i, 0))
_acc_spec = pl.BlockSpec((2, RB, D), lambda i: (0, i, 0))
_mat_spec = pl.BlockSpec((D, D), lambda i: (0, 0))
_vec_spec = pl.BlockSpec((1, D), lambda i: (0, 0))


def _tc_matmul(x, W1):
    return pl.pallas_call(
        _matmul_body,
        grid=(NB,),
        in_specs=[_row_spec, _mat_spec],
        out_specs=_row_spec,
        out_shape=jax.ShapeDtypeStruct((N, D), jnp.float32),
    )(x, W1)


def _tc_scale(degacc, h):
    return pl.pallas_call(
        _scale_body,
        grid=(NB,),
        in_specs=[_deg_spec, _row_spec],
        out_specs=_row_spec,
        out_shape=jax.ShapeDtypeStruct((N, D), jnp.float32),
    )(degacc, h)


def _tc_mid(degacc, acc, hs, b, Wn):
    return pl.pallas_call(
        _mid_body,
        grid=(NB,),
        in_specs=[_deg_spec, _acc_spec, _row_spec, _vec_spec, _mat_spec],
        out_specs=_row_spec,
        out_shape=jax.ShapeDtypeStruct((N, D), jnp.float32),
    )(degacc, acc, hs, b.reshape(1, D), Wn)


def _tc_last(degacc, acc, hs, b, Wlp, blp):
    return pl.pallas_call(
        _last_body,
        grid=(NB,),
        in_specs=[_deg_spec, _acc_spec, _row_spec, _vec_spec, _mat_spec,
                  _vec_spec],
        out_specs=_row_spec,
        out_shape=jax.ShapeDtypeStruct((N, D), jnp.float32),
    )(degacc, acc, hs, b.reshape(1, D), Wlp, blp)


@jax.jit
def kernel(x, edge_index, edge_attr, W1, b1, W2, b2, W3, b3, Wl, bl):
    e = edge_attr.shape[0]
    src = edge_index[0].astype(jnp.int32)
    dst = edge_index[1].astype(jnp.int32)
    w = edge_attr.reshape(-1).astype(jnp.float32)

    nchunk = -(-e // (NW * CHUNK))
    nchunk = -(-nchunk // GRP) * GRP
    epad = NW * CHUNK * nchunk
    pad = epad - e
    pidx = jnp.arange(pad, dtype=jnp.int32) % N
    src3 = jnp.concatenate([src, pidx]).reshape(NW, nchunk, CHUNK)
    dst3 = jnp.concatenate([dst, pidx]).reshape(NW, nchunk, CHUNK)
    w3 = jnp.concatenate([w, jnp.zeros((pad,), jnp.float32)]).reshape(
        NW, nchunk // 2, 2 * CHUNK
    )

    agg_call = _make_agg_kernel(nchunk)

    ones = jnp.ones((N, D), jnp.float32)
    h1 = _tc_matmul(x, W1)
    degacc = agg_call(ones, src3, dst3, w3)
    hs1 = _tc_scale(degacc, h1)
    acc1 = agg_call(hs1, src3, dst3, w3)
    hs2 = _tc_mid(degacc, acc1, hs1, b1, W2)
    acc2 = agg_call(hs2, src3, dst3, w3)
    hs3 = _tc_mid(degacc, acc2, hs2, b2, W3)
    acc3 = agg_call(hs3, src3, dst3, w3)

    Wlp = jnp.pad(Wl, ((0, 0), (0, D - Wl.shape[1])))
    blp = jnp.pad(bl, (0, D - bl.shape[0])).reshape(1, D)
    out = _tc_last(degacc, acc3, hs3, b3, Wlp, blp)
    return out[:, : Wl.shape[1]]


# speedup vs baseline: 1.1354x; 1.0113x over previous
import functools

import jax
import jax.numpy as jnp
from jax import lax
from jax.experimental import pallas as pl
from jax.experimental.pallas import tpu as pltpu
from jax.experimental.pallas import tpu_sc as plsc

N = 10000
D = 128
NC = 2
NS = 16
NW = NC * NS
CHUNK = 64
NPAD = 10240
RPT = NPAD // NS
RB = 2000
NB = N // RB

def _mesh():
    return plsc.VectorSubcoreMesh(
        core_axis_name="c", subcore_axis_name="s",
        num_cores=NC, num_subcores=NS,
    )


def _zero_rows(buf, nrows, width):
    zeros = jnp.zeros((16,), jnp.float32)

    def body(j, _):
        for s in range(width // 16):
            buf[j, pl.ds(s * 16, 16)] = zeros
        return 0

    lax.fori_loop(0, nrows, body, 0)


def _zero_acc_slice(zbuf, acc_s, base, nrows):
    full, rem = nrows // CHUNK, nrows % CHUNK
    for k in range(full):
        pltpu.sync_copy(zbuf, acc_s.at[pl.ds(base + k * CHUNK, CHUNK)])
    if rem:
        pltpu.sync_copy(
            zbuf.at[pl.ds(0, rem)], acc_s.at[pl.ds(base + full * CHUNK, rem)]
        )


NBUF = 4
NSRC = 8
GRP = 8


def _make_agg_kernel(nchunk):
    assert nchunk % GRP == 0
    ng = nchunk // GRP
    scratch = [pltpu.VMEM((nchunk // 2, 2 * CHUNK), jnp.float32)]
    scratch += [pltpu.VMEM((CHUNK, D), jnp.float32) for _ in range(NBUF)]
    scratch += [pltpu.VMEM((NSRC, CHUNK), jnp.int32),
                pltpu.VMEM((NBUF, CHUNK), jnp.int32)]
    scratch += [pltpu.VMEM_SHARED((NPAD, D), jnp.float32)]
    scratch += [pltpu.SemaphoreType.DMA
                for _ in range(NBUF + NBUF + NSRC + NBUF)]

    @functools.partial(
        pl.kernel,
        mesh=_mesh(),
        out_type=jax.ShapeDtypeStruct((NC, NPAD, D), jnp.float32),
        scratch_types=scratch,
    )
    def agg_kernel(hs_hbm, src_hbm, dst_hbm, w_hbm, out_hbm, w_v, *rest):
        rows = rest[:NBUF]
        srcb_a = rest[NBUF]
        dstb_a = rest[NBUF + 1]
        acc_s = rest[NBUF + 2]
        sems = rest[NBUF + 3:]
        gsem = sems[:NBUF]
        ssem = sems[NBUF:2 * NBUF]
        isems = sems[2 * NBUF:2 * NBUF + NSRC]
        isemd = sems[2 * NBUF + NSRC:]
        c = lax.axis_index("c")
        s = lax.axis_index("s")
        wid = s * NC + c

        def src_dma(j, si):
            pltpu.async_copy(src_hbm.at[wid, j], srcb_a.at[si], isems[si])

        def src_wait(j, si):
            pltpu.make_async_copy(
                src_hbm.at[wid, j], srcb_a.at[si], isems[si]).wait()

        def dst_dma(j, b):
            pltpu.async_copy(dst_hbm.at[wid, j], dstb_a.at[b], isemd[b])

        def dst_wait(j, b):
            pltpu.make_async_copy(
                dst_hbm.at[wid, j], dstb_a.at[b], isemd[b]).wait()

        def gather(b, si):
            pltpu.async_copy(hs_hbm.at[srcb_a.at[si]], rows[b], gsem[b])

        def gwait(b, si):
            pltpu.make_async_copy(
                hs_hbm.at[srcb_a.at[si]], rows[b], gsem[b]).wait()

        def scatter(b):
            pltpu.async_copy(rows[b], acc_s.at[dstb_a.at[b]], ssem[b],
                             add=True)

        def swait(b):
            pltpu.make_async_copy(
                rows[b], acc_s.at[dstb_a.at[b]], ssem[b]).wait()

        pltpu.sync_copy(w_hbm.at[wid], w_v)
        _zero_rows(rows[0], CHUNK, D)
        base = s * RPT
        _zero_acc_slice(rows[0], acc_s, base, RPT)
        for j in range(NBUF):
            src_dma(j, j)
        for j in range(2):
            dst_dma(j, j)
            src_wait(j, j)
            gather(j, j)
        plsc.subcore_barrier()

        def group(g, _):
            for p in range(GRP):
                j = g * GRP + p
                b = p % NBUF
                b2 = (p + 2) % NBUF
                s2 = (p + 2) % NSRC
                s4 = (p + 4) % NSRC

                def drain():
                    swait(b2)

                def prefetch_src():
                    src_dma(j + 4, s4)

                def prefetch_rows():
                    dst_dma(j + 2, b2)
                    src_wait(j + 2, s2)
                    gather(b2, s2)

                if p < 2:
                    pl.when(g >= 1)(drain)
                else:
                    drain()
                if p < NBUF:
                    prefetch_src()
                else:
                    pl.when(g <= ng - 2)(prefetch_src)
                if p < GRP - 2:
                    prefetch_rows()
                else:
                    pl.when(g <= ng - 2)(prefetch_rows)

                gwait(b, p % NSRC)

                jh = g * (GRP // 2) + p // 2
                woff = (p % 2) * CHUNK

                _dnums = lax.GatherDimensionNumbers(
                    offset_dims=(), collapsed_slice_dims=(0,),
                    start_index_map=(0,))

                def scale(gg, _):
                    wv16 = w_v[jh, pl.ds(woff + gg * 16, 16)]
                    for k in range(16):
                        kvec = jnp.full((16, 1), k, jnp.int32)
                        wrow = lax.gather(
                            wv16, kvec, _dnums, (1,),
                            mode=lax.GatherScatterMode.PROMISE_IN_BOUNDS)
                        jj = gg * 16 + k
                        for s8 in range(D // 16):
                            sl = pl.ds(s8 * 16, 16)
                            rows[b][jj, sl] = rows[b][jj, sl] * wrow
                    return 0

                lax.fori_loop(0, CHUNK // 16, scale, 0, unroll=2)
                dst_wait(j, b)
                scatter(b)
            return 0

        lax.fori_loop(0, ng, group, 0)
        swait((nchunk - 2) % NBUF)
        swait((nchunk - 1) % NBUF)
        plsc.subcore_barrier()
        pltpu.sync_copy(
            acc_s.at[pl.ds(base, RPT)], out_hbm.at[c, pl.ds(base, RPT)]
        )

    return agg_kernel


def _dis_block(deg_ref):
    deg = deg_ref[0, :, :1] + deg_ref[1, :, :1] + 1.0
    return jnp.where(deg > 0, 1.0 / jnp.sqrt(deg), 0.0)


def _matmul_body(x_ref, w_ref, o_ref):
    o_ref[...] = jnp.dot(
        x_ref[...], w_ref[...], preferred_element_type=jnp.float32
    )


def _scale_body(deg_ref, h_ref, o_ref):
    o_ref[...] = h_ref[...] * _dis_block(deg_ref)


def _mid_body(deg_ref, acc_ref, hs_ref, b_ref, w_ref, o_ref):
    dis = _dis_block(deg_ref)
    t = (acc_ref[0] + acc_ref[1] + hs_ref[...]) * dis + b_ref[...]
    z = jnp.maximum(t, 0.0)
    o_ref[...] = (
        jnp.dot(z, w_ref[...], preferred_element_type=jnp.float32) * dis
    )


def _last_body(deg_ref, acc_ref, hs_ref, b_ref, w_ref, bl_ref, o_ref):
    dis = _dis_block(deg_ref)
    t = (acc_ref[0] + acc_ref[1] + hs_ref[...]) * dis + b_ref[...]
    z = jnp.maximum(t, 0.0)
    o_ref[...] = (
        jnp.dot(z, w_ref[...], preferred_element_type=jnp.float32)
        + bl_ref[...]
    )


_deg_spec = pl.BlockSpec((2, RB, D), lambda i: (0, i, 0))
_row_spec = pl.BlockSpec((RB, D), lambda i: (i, 0))
_acc_spec = pl.BlockSpec((2, RB, D), lambda i: (0, i, 0))
_mat_spec = pl.BlockSpec((D, D), lambda i: (0, 0))
_vec_spec = pl.BlockSpec((1, D), lambda i: (0, 0))


def _tc_matmul(x, W1):
    return pl.pallas_call(
        _matmul_body,
        grid=(NB,),
        in_specs=[_row_spec, _mat_spec],
        out_specs=_row_spec,
        out_shape=jax.ShapeDtypeStruct((N, D), jnp.float32),
    )(x, W1)


def _tc_scale(degacc, h):
    return pl.pallas_call(
        _scale_body,
        grid=(NB,),
        in_specs=[_deg_spec, _row_spec],
        out_specs=_row_spec,
        out_shape=jax.ShapeDtypeStruct((N, D), jnp.float32),
    )(degacc, h)


def _tc_mid(degacc, acc, hs, b, Wn):
    return pl.pallas_call(
        _mid_body,
        grid=(NB,),
        in_specs=[_deg_spec, _acc_spec, _row_spec, _vec_spec, _mat_spec],
        out_specs=_row_spec,
        out_shape=jax.ShapeDtypeStruct((N, D), jnp.float32),
    )(degacc, acc, hs, b.reshape(1, D), Wn)


def _tc_last(degacc, acc, hs, b, Wlp, blp):
    return pl.pallas_call(
        _last_body,
        grid=(NB,),
        in_specs=[_deg_spec, _acc_spec, _row_spec, _vec_spec, _mat_spec,
                  _vec_spec],
        out_specs=_row_spec,
        out_shape=jax.ShapeDtypeStruct((N, D), jnp.float32),
    )(degacc, acc, hs, b.reshape(1, D), Wlp, blp)


@jax.jit
def kernel(x, edge_index, edge_attr, W1, b1, W2, b2, W3, b3, Wl, bl):
    e = edge_attr.shape[0]
    src = edge_index[0].astype(jnp.int32)
    dst = edge_index[1].astype(jnp.int32)
    w = edge_attr.reshape(-1).astype(jnp.float32)

    nchunk = -(-e // (NW * CHUNK))
    nchunk = -(-nchunk // GRP) * GRP
    epad = NW * CHUNK * nchunk
    pad = epad - e
    pidx = jnp.arange(pad, dtype=jnp.int32) % N
    src3 = jnp.concatenate([src, pidx]).reshape(NW, nchunk, CHUNK)
    dst3 = jnp.concatenate([dst, pidx]).reshape(NW, nchunk, CHUNK)
    w3 = jnp.concatenate([w, jnp.zeros((pad,), jnp.float32)]).reshape(
        NW, nchunk // 2, 2 * CHUNK
    )

    agg_call = _make_agg_kernel(nchunk)

    ones = jnp.ones((N, D), jnp.float32)
    h1 = _tc_matmul(x, W1)
    degacc = agg_call(ones, src3, dst3, w3)
    hs1 = _tc_scale(degacc, h1)
    acc1 = agg_call(hs1, src3, dst3, w3)
    hs2 = _tc_mid(degacc, acc1, hs1, b1, W2)
    acc2 = agg_call(hs2, src3, dst3, w3)
    hs3 = _tc_mid(degacc, acc2, hs2, b2, W3)
    acc3 = agg_call(hs3, src3, dst3, w3)

    Wlp = jnp.pad(Wl, ((0, 0), (0, D - Wl.shape[1])))
    blp = jnp.pad(bl, (0, D - bl.shape[0])).reshape(1, D)
    out = _tc_last(degacc, acc3, hs3, b3, Wlp, blp)
    return out[:, : Wl.shape[1]]


# speedup vs baseline: 1.1621x; 1.0235x over previous
import functools

import jax
import jax.numpy as jnp
from jax import lax
from jax.experimental import pallas as pl
from jax.experimental.pallas import tpu as pltpu
from jax.experimental.pallas import tpu_sc as plsc

N = 10000
D = 128
NC = 2
NS = 16
NW = NC * NS
CHUNK = 64
NPAD = 10240
RPT = NPAD // NS
RB = 2000
NB = N // RB

def _mesh():
    return plsc.VectorSubcoreMesh(
        core_axis_name="c", subcore_axis_name="s",
        num_cores=NC, num_subcores=NS,
    )


def _zero_rows(buf, nrows, width):
    zeros = jnp.zeros((16,), jnp.float32)

    def body(j, _):
        for s in range(width // 16):
            buf[j, pl.ds(s * 16, 16)] = zeros
        return 0

    lax.fori_loop(0, nrows, body, 0)


def _zero_acc_slice(zbuf, acc_s, base, nrows):
    full, rem = nrows // CHUNK, nrows % CHUNK
    for k in range(full):
        pltpu.sync_copy(zbuf, acc_s.at[pl.ds(base + k * CHUNK, CHUNK)])
    if rem:
        pltpu.sync_copy(
            zbuf.at[pl.ds(0, rem)], acc_s.at[pl.ds(base + full * CHUNK, rem)]
        )


NBUF = 4
NSRC = 8
GRP = 8


def _make_agg_kernel(nchunk):
    assert nchunk % GRP == 0
    ng = nchunk // GRP
    scratch = [pltpu.VMEM((nchunk // 2, 2 * CHUNK), jnp.float32)]
    scratch += [pltpu.VMEM((CHUNK, D), jnp.float32) for _ in range(NBUF)]
    scratch += [pltpu.VMEM((NSRC, CHUNK), jnp.int32),
                pltpu.VMEM((NBUF, CHUNK), jnp.int32),
                pltpu.VMEM((8, 128), jnp.float32)]
    scratch += [pltpu.VMEM_SHARED((NPAD, D), jnp.float32)]
    scratch += [pltpu.SemaphoreType.DMA
                for _ in range(NBUF + NBUF + NSRC + NBUF)]

    @functools.partial(
        pl.kernel,
        mesh=_mesh(),
        out_type=jax.ShapeDtypeStruct((NC, NPAD, D), jnp.float32),
        scratch_types=scratch,
    )
    def agg_kernel(hs_hbm, src_hbm, dst_hbm, w_hbm, flag_hbm, out_hbm,
                   w_v, *rest):
        rows = rest[:NBUF]
        srcb_a = rest[NBUF]
        dstb_a = rest[NBUF + 1]
        flag_v = rest[NBUF + 2]
        acc_s = rest[NBUF + 3]
        sems = rest[NBUF + 4:]
        gsem = sems[:NBUF]
        ssem = sems[NBUF:2 * NBUF]
        isems = sems[2 * NBUF:2 * NBUF + NSRC]
        isemd = sems[2 * NBUF + NSRC:]
        c = lax.axis_index("c")
        s = lax.axis_index("s")
        wid = s * NC + c

        def src_dma(j, si):
            pltpu.async_copy(src_hbm.at[wid, j], srcb_a.at[si], isems[si])

        def src_wait(j, si):
            pltpu.make_async_copy(
                src_hbm.at[wid, j], srcb_a.at[si], isems[si]).wait()

        def dst_dma(j, b):
            pltpu.async_copy(dst_hbm.at[wid, j], dstb_a.at[b], isemd[b])

        def dst_wait(j, b):
            pltpu.make_async_copy(
                dst_hbm.at[wid, j], dstb_a.at[b], isemd[b]).wait()

        def gather(b, si):
            pltpu.async_copy(hs_hbm.at[srcb_a.at[si]], rows[b], gsem[b])

        def gwait(b, si):
            pltpu.make_async_copy(
                hs_hbm.at[srcb_a.at[si]], rows[b], gsem[b]).wait()

        def scatter(b):
            pltpu.async_copy(rows[b], acc_s.at[dstb_a.at[b]], ssem[b],
                             add=True)

        def swait(b):
            pltpu.make_async_copy(
                rows[b], acc_s.at[dstb_a.at[b]], ssem[b]).wait()

        pltpu.sync_copy(w_hbm.at[wid], w_v)
        pltpu.sync_copy(flag_hbm, flag_v)
        gmode = flag_v[c * 0, pl.ds(0, 16)][0] > 0.5
        _zero_rows(rows[0], CHUNK, D)
        base = s * RPT
        _zero_acc_slice(rows[0], acc_s, base, RPT)

        def prime():
            for j in range(NBUF):
                src_dma(j, j)
            for j in range(2):
                src_wait(j, j)
                gather(j, j)

        pl.when(gmode)(prime)
        for j in range(2):
            dst_dma(j, j)
        plsc.subcore_barrier()
        ng_main = jnp.where(gmode, ng, 0)
        ng_deg = jnp.where(gmode, 0, ng)

        def group(g, _):
            for p in range(GRP):
                j = g * GRP + p
                b = p % NBUF
                b2 = (p + 2) % NBUF
                s2 = (p + 2) % NSRC
                s4 = (p + 4) % NSRC

                def drain():
                    swait(b2)

                def prefetch_src():
                    src_dma(j + 4, s4)

                def prefetch_rows():
                    dst_dma(j + 2, b2)
                    src_wait(j + 2, s2)
                    gather(b2, s2)

                if p < 2:
                    pl.when(g >= 1)(drain)
                else:
                    drain()
                if p < NBUF:
                    prefetch_src()
                else:
                    pl.when(g <= ng - 2)(prefetch_src)
                if p < GRP - 2:
                    prefetch_rows()
                else:
                    pl.when(g <= ng - 2)(prefetch_rows)

                gwait(b, p % NSRC)

                jh = g * (GRP // 2) + p // 2
                woff = (p % 2) * CHUNK

                _dnums = lax.GatherDimensionNumbers(
                    offset_dims=(), collapsed_slice_dims=(0,),
                    start_index_map=(0,))

                def scale(gg, _):
                    wv16 = w_v[jh, pl.ds(woff + gg * 16, 16)]
                    for k in range(16):
                        kvec = jnp.full((16, 1), k, jnp.int32)
                        wrow = lax.gather(
                            wv16, kvec, _dnums, (1,),
                            mode=lax.GatherScatterMode.PROMISE_IN_BOUNDS)
                        jj = gg * 16 + k
                        for s8 in range(D // 16):
                            sl = pl.ds(s8 * 16, 16)
                            rows[b][jj, sl] = rows[b][jj, sl] * wrow
                    return 0

                lax.fori_loop(0, CHUNK // 16, scale, 0, unroll=2)
                dst_wait(j, b)
                scatter(b)
            return 0

        lax.fori_loop(0, ng_main, group, 0)

        def dgroup(g, _):
            for p in range(GRP):
                j = g * GRP + p
                b = p % NBUF
                b2 = (p + 2) % NBUF

                def drain():
                    swait(b2)

                def prefetch_dst():
                    dst_dma(j + 2, b2)

                if p < 2:
                    pl.when(g >= 1)(drain)
                else:
                    drain()
                if p < GRP - 2:
                    prefetch_dst()
                else:
                    pl.when(g <= ng - 2)(prefetch_dst)

                jh = g * (GRP // 2) + p // 2
                woff = (p % 2) * CHUNK

                def fill(gg, _):
                    wv16 = w_v[jh, pl.ds(woff + gg * 16, 16)]
                    for k in range(16):
                        wrow = jnp.full((16,), wv16[k], jnp.float32)
                        jj = gg * 16 + k
                        for s8 in range(D // 16):
                            rows[b][jj, pl.ds(s8 * 16, 16)] = wrow
                    return 0

                lax.fori_loop(0, CHUNK // 16, fill, 0, unroll=2)
                dst_wait(j, b)
                scatter(b)
            return 0

        lax.fori_loop(0, ng_deg, dgroup, 0)
        swait((nchunk - 2) % NBUF)
        swait((nchunk - 1) % NBUF)
        plsc.subcore_barrier()
        pltpu.sync_copy(
            acc_s.at[pl.ds(base, RPT)], out_hbm.at[c, pl.ds(base, RPT)]
        )

    return agg_kernel


def _dis_block(deg_ref):
    deg = deg_ref[0, :, :1] + deg_ref[1, :, :1] + 1.0
    return jnp.where(deg > 0, 1.0 / jnp.sqrt(deg), 0.0)


def _matmul_body(x_ref, w_ref, o_ref):
    o_ref[...] = jnp.dot(
        x_ref[...], w_ref[...], preferred_element_type=jnp.float32
    )


def _scale_body(deg_ref, h_ref, o_ref):
    o_ref[...] = h_ref[...] * _dis_block(deg_ref)


def _mid_body(deg_ref, acc_ref, hs_ref, b_ref, w_ref, o_ref):
    dis = _dis_block(deg_ref)
    t = (acc_ref[0] + acc_ref[1] + hs_ref[...]) * dis + b_ref[...]
    z = jnp.maximum(t, 0.0)
    o_ref[...] = (
        jnp.dot(z, w_ref[...], preferred_element_type=jnp.float32) * dis
    )


def _last_body(deg_ref, acc_ref, hs_ref, b_ref, w_ref, bl_ref, o_ref):
    dis = _dis_block(deg_ref)
    t = (acc_ref[0] + acc_ref[1] + hs_ref[...]) * dis + b_ref[...]
    z = jnp.maximum(t, 0.0)
    o_ref[...] = (
        jnp.dot(z, w_ref[...], preferred_element_type=jnp.float32)
        + bl_ref[...]
    )


_deg_spec = pl.BlockSpec((2, RB, D), lambda i: (0, i, 0))
_row_spec = pl.BlockSpec((RB, D), lambda i: (i, 0))
_acc_spec = pl.BlockSpec((2, RB, D), lambda i: (0, i, 0))
_mat_spec = pl.BlockSpec((D, D), lambda i: (0, 0))
_vec_spec = pl.BlockSpec((1, D), lambda i: (0, 0))


def _tc_matmul(x, W1):
    return pl.pallas_call(
        _matmul_body,
        grid=(NB,),
        in_specs=[_row_spec, _mat_spec],
        out_specs=_row_spec,
        out_shape=jax.ShapeDtypeStruct((N, D), jnp.float32),
    )(x, W1)


def _tc_scale(degacc, h):
    return pl.pallas_call(
        _scale_body,
        grid=(NB,),
        in_specs=[_deg_spec, _row_spec],
        out_specs=_row_spec,
        out_shape=jax.ShapeDtypeStruct((N, D), jnp.float32),
    )(degacc, h)


def _tc_mid(degacc, acc, hs, b, Wn):
    return pl.pallas_call(
        _mid_body,
        grid=(NB,),
        in_specs=[_deg_spec, _acc_spec, _row_spec, _vec_spec, _mat_spec],
        out_specs=_row_spec,
        out_shape=jax.ShapeDtypeStruct((N, D), jnp.float32),
    )(degacc, acc, hs, b.reshape(1, D), Wn)


def _tc_last(degacc, acc, hs, b, Wlp, blp):
    return pl.pallas_call(
        _last_body,
        grid=(NB,),
        in_specs=[_deg_spec, _acc_spec, _row_spec, _vec_spec, _mat_spec,
                  _vec_spec],
        out_specs=_row_spec,
        out_shape=jax.ShapeDtypeStruct((N, D), jnp.float32),
    )(degacc, acc, hs, b.reshape(1, D), Wlp, blp)


@jax.jit
def kernel(x, edge_index, edge_attr, W1, b1, W2, b2, W3, b3, Wl, bl):
    e = edge_attr.shape[0]
    src = edge_index[0].astype(jnp.int32)
    dst = edge_index[1].astype(jnp.int32)
    w = edge_attr.reshape(-1).astype(jnp.float32)

    nchunk = -(-e // (NW * CHUNK))
    nchunk = -(-nchunk // GRP) * GRP
    epad = NW * CHUNK * nchunk
    pad = epad - e
    pidx = jnp.arange(pad, dtype=jnp.int32) % N
    src3 = jnp.concatenate([src, pidx]).reshape(NW, nchunk, CHUNK)
    dst3 = jnp.concatenate([dst, pidx]).reshape(NW, nchunk, CHUNK)
    w3 = jnp.concatenate([w, jnp.zeros((pad,), jnp.float32)]).reshape(
        NW, nchunk // 2, 2 * CHUNK
    )

    agg_call = _make_agg_kernel(nchunk)

    fyes = jnp.ones((8, 128), jnp.float32)
    fno = jnp.zeros((8, 128), jnp.float32)
    h1 = _tc_matmul(x, W1)
    degacc = agg_call(x, src3, dst3, w3, fno)
    hs1 = _tc_scale(degacc, h1)
    acc1 = agg_call(hs1, src3, dst3, w3, fyes)
    hs2 = _tc_mid(degacc, acc1, hs1, b1, W2)
    acc2 = agg_call(hs2, src3, dst3, w3, fyes)
    hs3 = _tc_mid(degacc, acc2, hs2, b2, W3)
    acc3 = agg_call(hs3, src3, dst3, w3, fyes)

    Wlp = jnp.pad(Wl, ((0, 0), (0, D - Wl.shape[1])))
    blp = jnp.pad(bl, (0, D - bl.shape[0])).reshape(1, D)
    out = _tc_last(degacc, acc3, hs3, b3, Wlp, blp)
    return out[:, : Wl.shape[1]]


# speedup vs baseline: 1.2212x; 1.0508x over previous
import functools

import jax
import jax.numpy as jnp
from jax import lax
from jax.experimental import pallas as pl
from jax.experimental.pallas import tpu as pltpu
from jax.experimental.pallas import tpu_sc as plsc

N = 10000
D = 128
NC = 2
NS = 16
NW = NC * NS
CHUNK = 64
NPAD = 10240
RPT = NPAD // NS
RB = 2000
NB = N // RB

def _mesh():
    return plsc.VectorSubcoreMesh(
        core_axis_name="c", subcore_axis_name="s",
        num_cores=NC, num_subcores=NS,
    )


def _zero_rows(buf, nrows, width):
    zeros = jnp.zeros((16,), jnp.float32)

    def body(j, _):
        for s in range(width // 16):
            buf[j, pl.ds(s * 16, 16)] = zeros
        return 0

    lax.fori_loop(0, nrows, body, 0)


def _zero_acc_slice(zbuf, acc_s, base, nrows):
    full, rem = nrows // CHUNK, nrows % CHUNK
    for k in range(full):
        pltpu.sync_copy(zbuf, acc_s.at[pl.ds(base + k * CHUNK, CHUNK)])
    if rem:
        pltpu.sync_copy(
            zbuf.at[pl.ds(0, rem)], acc_s.at[pl.ds(base + full * CHUNK, rem)]
        )


NBUF = 4
NSRC = 8
GRP = 8


def _make_agg_kernel(nchunk):
    assert nchunk % GRP == 0
    ng = nchunk // GRP
    scratch = [pltpu.VMEM((nchunk // 2, 2 * CHUNK), jnp.float32)]
    scratch += [pltpu.VMEM((CHUNK, D), jnp.float32) for _ in range(NBUF)]
    scratch += [pltpu.VMEM((NSRC, CHUNK), jnp.int32),
                pltpu.VMEM((NBUF, CHUNK), jnp.int32),
                pltpu.VMEM((8, 128), jnp.float32)]
    scratch += [pltpu.VMEM_SHARED((NPAD, D), jnp.float32)]
    scratch += [pltpu.SemaphoreType.DMA
                for _ in range(NBUF + NBUF + NSRC + NBUF)]

    @functools.partial(
        pl.kernel,
        mesh=_mesh(),
        out_type=jax.ShapeDtypeStruct((NC, NPAD, D), jnp.float32),
        scratch_types=scratch,
    )
    def agg_kernel(hs_hbm, src_hbm, dst_hbm, w_hbm, flag_hbm, out_hbm,
                   w_v, *rest):
        rows = rest[:NBUF]
        srcb_a = rest[NBUF]
        dstb_a = rest[NBUF + 1]
        flag_v = rest[NBUF + 2]
        acc_s = rest[NBUF + 3]
        sems = rest[NBUF + 4:]
        gsem = sems[:NBUF]
        ssem = sems[NBUF:2 * NBUF]
        isems = sems[2 * NBUF:2 * NBUF + NSRC]
        isemd = sems[2 * NBUF + NSRC:]
        c = lax.axis_index("c")
        s = lax.axis_index("s")
        wid = s * NC + c

        def src_dma(j, si):
            pltpu.async_copy(src_hbm.at[wid, j], srcb_a.at[si], isems[si])

        def src_wait(j, si):
            pltpu.make_async_copy(
                src_hbm.at[wid, j], srcb_a.at[si], isems[si]).wait()

        def dst_dma(j, b):
            pltpu.async_copy(dst_hbm.at[wid, j], dstb_a.at[b], isemd[b])

        def dst_wait(j, b):
            pltpu.make_async_copy(
                dst_hbm.at[wid, j], dstb_a.at[b], isemd[b]).wait()

        def gather(b, si):
            pltpu.async_copy(hs_hbm.at[srcb_a.at[si]], rows[b], gsem[b])

        def gwait(b, si):
            pltpu.make_async_copy(
                hs_hbm.at[srcb_a.at[si]], rows[b], gsem[b]).wait()

        def scatter(b):
            pltpu.async_copy(rows[b], acc_s.at[dstb_a.at[b]], ssem[b],
                             add=True)

        def swait(b):
            pltpu.make_async_copy(
                rows[b], acc_s.at[dstb_a.at[b]], ssem[b]).wait()

        pltpu.sync_copy(w_hbm.at[wid], w_v)
        pltpu.sync_copy(flag_hbm, flag_v)
        gmode = flag_v[c * 0, pl.ds(0, 16)][0] > 0.5
        _zero_rows(rows[0], CHUNK, D)
        base = s * RPT
        _zero_acc_slice(rows[0], acc_s, base, RPT)

        def prime():
            for j in range(NBUF):
                src_dma(j, j)
            for j in range(2):
                src_wait(j, j)
                gather(j, j)

        pl.when(gmode)(prime)
        for j in range(2):
            dst_dma(j, j)
        plsc.subcore_barrier()
        ng_main = jnp.where(gmode, ng, 0)
        ng_deg = jnp.where(gmode, 0, ng)

        def group(g, _):
            for p in range(GRP):
                j = g * GRP + p
                b = p % NBUF
                b2 = (p + 2) % NBUF
                s2 = (p + 2) % NSRC
                s4 = (p + 4) % NSRC

                def drain():
                    swait(b2)

                def prefetch_src():
                    src_dma(j + 4, s4)

                def prefetch_rows():
                    dst_dma(j + 2, b2)
                    src_wait(j + 2, s2)
                    gather(b2, s2)

                if p < 2:
                    pl.when(g >= 1)(drain)
                else:
                    drain()
                if p < NBUF:
                    prefetch_src()
                else:
                    pl.when(g <= ng - 2)(prefetch_src)
                if p < GRP - 2:
                    prefetch_rows()
                else:
                    pl.when(g <= ng - 2)(prefetch_rows)

                gwait(b, p % NSRC)

                jh = g * (GRP // 2) + p // 2
                woff = (p % 2) * CHUNK

                _dnums = lax.GatherDimensionNumbers(
                    offset_dims=(), collapsed_slice_dims=(0,),
                    start_index_map=(0,))

                def scale(gg, _):
                    wv16 = w_v[jh, pl.ds(woff + gg * 16, 16)]
                    for k in range(16):
                        kvec = jnp.full((16, 1), k, jnp.int32)
                        wrow = lax.gather(
                            wv16, kvec, _dnums, (1,),
                            mode=lax.GatherScatterMode.PROMISE_IN_BOUNDS)
                        jj = gg * 16 + k
                        for s8 in range(D // 16):
                            sl = pl.ds(s8 * 16, 16)
                            rows[b][jj, sl] = rows[b][jj, sl] * wrow
                    return 0

                lax.fori_loop(0, CHUNK // 16, scale, 0, unroll=2)
                dst_wait(j, b)
                scatter(b)
            return 0

        lax.fori_loop(0, ng_main, group, 0)

        def dgroup(g, _):
            for p in range(GRP):
                j = g * GRP + p
                b = p % NBUF
                b2 = (p + 2) % NBUF

                def drain():
                    swait(b2)

                def prefetch_dst():
                    dst_dma(j + 2, b2)

                if p < 2:
                    pl.when(g >= 1)(drain)
                else:
                    drain()
                if p < GRP - 2:
                    prefetch_dst()
                else:
                    pl.when(g <= ng - 2)(prefetch_dst)

                jh = g * (GRP // 2) + p // 2
                woff = (p % 2) * CHUNK

                def fill(gg, _):
                    wv16 = w_v[jh, pl.ds(woff + gg * 16, 16)]
                    for k in range(16):
                        wrow = jnp.full((16,), wv16[k], jnp.float32)
                        rows[b][gg * 16 + k, pl.ds(0, 16)] = wrow
                    return 0

                lax.fori_loop(0, CHUNK // 16, fill, 0, unroll=2)
                dst_wait(j, b)
                scatter(b)
            return 0

        lax.fori_loop(0, ng_deg, dgroup, 0)
        swait((nchunk - 2) % NBUF)
        swait((nchunk - 1) % NBUF)
        plsc.subcore_barrier()
        pltpu.sync_copy(
            acc_s.at[pl.ds(base, RPT)], out_hbm.at[c, pl.ds(base, RPT)]
        )

    return agg_kernel


def _dis_block(deg_ref):
    deg = deg_ref[0, :, :1] + deg_ref[1, :, :1] + 1.0
    return jnp.where(deg > 0, 1.0 / jnp.sqrt(deg), 0.0)


def _matmul_body(x_ref, w_ref, o_ref):
    o_ref[...] = jnp.dot(
        x_ref[...], w_ref[...], preferred_element_type=jnp.float32
    )


def _scale_body(deg_ref, h_ref, o_ref):
    o_ref[...] = h_ref[...] * _dis_block(deg_ref)


def _mid_body(deg_ref, acc_ref, hs_ref, b_ref, w_ref, o_ref):
    dis = _dis_block(deg_ref)
    t = (acc_ref[0] + acc_ref[1] + hs_ref[...]) * dis + b_ref[...]
    z = jnp.maximum(t, 0.0)
    o_ref[...] = (
        jnp.dot(z, w_ref[...], preferred_element_type=jnp.float32) * dis
    )


def _last_body(deg_ref, acc_ref, hs_ref, b_ref, w_ref, bl_ref, o_ref):
    dis = _dis_block(deg_ref)
    t = (acc_ref[0] + acc_ref[1] + hs_ref[...]) * dis + b_ref[...]
    z = jnp.maximum(t, 0.0)
    o_ref[...] = (
        jnp.dot(z, w_ref[...], preferred_element_type=jnp.float32)
        + bl_ref[...]
    )


_deg_spec = pl.BlockSpec((2, RB, D), lambda i: (0, i, 0))
_row_spec = pl.BlockSpec((RB, D), lambda i: (i, 0))
_acc_spec = pl.BlockSpec((2, RB, D), lambda i: (0, i, 0))
_mat_spec = pl.BlockSpec((D, D), lambda i: (0, 0))
_vec_spec = pl.BlockSpec((1, D), lambda i: (0, 0))


def _tc_matmul(x, W1):
    return pl.pallas_call(
        _matmul_body,
        grid=(NB,),
        in_specs=[_row_spec, _mat_spec],
        out_specs=_row_spec,
        out_shape=jax.ShapeDtypeStruct((N, D), jnp.float32),
    )(x, W1)


def _tc_scale(degacc, h):
    return pl.pallas_call(
        _scale_body,
        grid=(NB,),
        in_specs=[_deg_spec, _row_spec],
        out_specs=_row_spec,
        out_shape=jax.ShapeDtypeStruct((N, D), jnp.float32),
    )(degacc, h)


def _tc_mid(degacc, acc, hs, b, Wn):
    return pl.pallas_call(
        _mid_body,
        grid=(NB,),
        in_specs=[_deg_spec, _acc_spec, _row_spec, _vec_spec, _mat_spec],
        out_specs=_row_spec,
        out_shape=jax.ShapeDtypeStruct((N, D), jnp.float32),
    )(degacc, acc, hs, b.reshape(1, D), Wn)


def _tc_last(degacc, acc, hs, b, Wlp, blp):
    return pl.pallas_call(
        _last_body,
        grid=(NB,),
        in_specs=[_deg_spec, _acc_spec, _row_spec, _vec_spec, _mat_spec,
                  _vec_spec],
        out_specs=_row_spec,
        out_shape=jax.ShapeDtypeStruct((N, D), jnp.float32),
    )(degacc, acc, hs, b.reshape(1, D), Wlp, blp)


@jax.jit
def kernel(x, edge_index, edge_attr, W1, b1, W2, b2, W3, b3, Wl, bl):
    e = edge_attr.shape[0]
    src = edge_index[0].astype(jnp.int32)
    dst = edge_index[1].astype(jnp.int32)
    w = edge_attr.reshape(-1).astype(jnp.float32)

    nchunk = -(-e // (NW * CHUNK))
    nchunk = -(-nchunk // GRP) * GRP
    epad = NW * CHUNK * nchunk
    pad = epad - e
    pidx = jnp.arange(pad, dtype=jnp.int32) % N
    src3 = jnp.concatenate([src, pidx]).reshape(NW, nchunk, CHUNK)
    dst3 = jnp.concatenate([dst, pidx]).reshape(NW, nchunk, CHUNK)
    w3 = jnp.concatenate([w, jnp.zeros((pad,), jnp.float32)]).reshape(
        NW, nchunk // 2, 2 * CHUNK
    )

    agg_call = _make_agg_kernel(nchunk)

    fyes = jnp.ones((8, 128), jnp.float32)
    fno = jnp.zeros((8, 128), jnp.float32)
    h1 = _tc_matmul(x, W1)
    degacc = agg_call(x, src3, dst3, w3, fno)
    hs1 = _tc_scale(degacc, h1)
    acc1 = agg_call(hs1, src3, dst3, w3, fyes)
    hs2 = _tc_mid(degacc, acc1, hs1, b1, W2)
    acc2 = agg_call(hs2, src3, dst3, w3, fyes)
    hs3 = _tc_mid(degacc, acc2, hs2, b2, W3)
    acc3 = agg_call(hs3, src3, dst3, w3, fyes)

    Wlp = jnp.pad(Wl, ((0, 0), (0, D - Wl.shape[1])))
    blp = jnp.pad(bl, (0, D - bl.shape[0])).reshape(1, D)
    out = _tc_last(degacc, acc3, hs3, b3, Wlp, blp)
    return out[:, : Wl.shape[1]]
